# baseline (speedup 1.0000x reference)
import functools

import jax
import jax.numpy as jnp
from jax import lax
from jax.experimental import pallas as pl
from jax.experimental.pallas import tpu as pltpu
from jax.experimental.pallas import tpu_sc as plsc


def _pick_chunk(epw):
    for ch in (128, 112, 96, 80, 64, 48, 32, 16):
        if epw % ch == 0:
            return ch
    raise ValueError(f"edges per worker {epw} not divisible by 16")


def _memset_rows(ref, nrows, ncols, value):
    vec = jnp.full((16,), value, jnp.float32)

    def body(i, _):
        for j in range(ncols // 16):
            ref[i, pl.ds(j * 16, 16)] = vec
        return 0

    lax.fori_loop(0, nrows, body, 0)


def _memset_vec(ref, length, value):
    vec = jnp.full((16,), value, jnp.float32)
    for k in range(length // 16):
        ref[pl.ds(k * 16, 16)] = vec


def _copy_idx(src, dst, base, ch):
    for k in range(ch // 16):
        dst[pl.ds(k * 16, 16)] = src[pl.ds(base + k * 16, 16)]


def _zero_stripe(acc_sh, zeros_v, base_row, nrows, ch):
    nfull = nrows // ch
    rem = nrows - nfull * ch

    def body(i, _):
        pltpu.sync_copy(zeros_v, acc_sh.at[pl.ds(base_row + i * ch, ch)])
        return 0

    lax.fori_loop(0, nfull, body, 0)
    if rem:
        pltpu.sync_copy(zeros_v.at[pl.ds(0, rem)],
                        acc_sh.at[pl.ds(base_row + nfull * ch, rem)])


@functools.lru_cache(maxsize=None)
def _make_degree(n, e):
    info = plsc.get_sparse_core_info()
    nc, ns = info.num_cores, info.num_subcores
    nw = nc * ns
    epw = e // nw
    ch = _pick_chunk(epw)
    nchunk = epw // ch
    n_pad = pl.cdiv(n + 1, ns * 128) * ns * 128
    rpt = n_pad // ns
    mesh = plsc.VectorSubcoreMesh(core_axis_name="c", subcore_axis_name="s")

    @functools.partial(
        pl.kernel,
        out_type=jax.ShapeDtypeStruct((nc, n_pad), jnp.float32),
        mesh=mesh,
        scratch_types=[
            pltpu.VMEM((epw,), jnp.int32),
            pltpu.VMEM((ch,), jnp.int32),
            pltpu.VMEM((ch,), jnp.float32),
            pltpu.VMEM((ch,), jnp.float32),
            pltpu.VMEM_SHARED((n_pad,), jnp.float32),
        ],
    )
    def kern(row_hbm, out_hbm, row_all, row_v, ones_v, zeros_v, acc_sh):
        cid = lax.axis_index("c")
        sid = lax.axis_index("s")
        wid = sid * nc + cid

        _memset_vec(ones_v, ch, 1.0)
        _memset_vec(zeros_v, ch, 0.0)
        base_row = sid * rpt

        def zbody(i, _):
            pltpu.sync_copy(zeros_v, acc_sh.at[pl.ds(base_row + i * ch, ch)])
            return 0

        lax.fori_loop(0, rpt // ch, zbody, 0)
        if rpt % ch:
            pltpu.sync_copy(zeros_v.at[pl.ds(0, rpt % ch)],
                            acc_sh.at[pl.ds(base_row + (rpt // ch) * ch,
                                            rpt % ch)])
        pltpu.sync_copy(row_hbm.at[pl.ds(wid * epw, epw)], row_all)
        plsc.subcore_barrier()

        def chunk(i, _):
            _copy_idx(row_all, row_v, i * ch, ch)
            pltpu.sync_copy(ones_v, acc_sh.at[row_v], add=True)
            return 0

        lax.fori_loop(0, nchunk, chunk, 0)
        plsc.subcore_barrier()

        pltpu.sync_copy(acc_sh.at[pl.ds(base_row, rpt)],
                        out_hbm.at[cid, pl.ds(base_row, rpt)])

    return kern


_NBUF = 5


@functools.lru_cache(maxsize=None)
def _make_edge_aggregate(n, e, d):
    info = plsc.get_sparse_core_info()
    nc, ns = info.num_cores, info.num_subcores
    nw = nc * ns
    epw = e // nw
    n_pad = pl.cdiv(n + 1, ns * 8) * ns * 8
    rpt = n_pad // ns
    tile_budget = (2097151 - n_pad * d) // ns - 2048
    ch, nb = None, 1
    for c_ in (128, 112, 96, 80, 64, 48, 32, 16):
        if epw % c_:
            continue
        nb_ = _NBUF if (epw // c_) % _NBUF == 0 else 1
        if 2 * epw + 2 * nb_ * c_ + nb_ * c_ * d <= tile_budget:
            if nb_ > 1 or ch is None:
                ch, nb = c_, nb_
            if nb_ > 1:
                break
    if ch is None:
        ch, nb = _pick_chunk(epw), 1
    nchunk = epw // ch
    mesh = plsc.VectorSubcoreMesh(core_axis_name="c", subcore_axis_name="s")

    @functools.partial(
        pl.kernel,
        out_type=jax.ShapeDtypeStruct((nc, n_pad, d), jnp.float32),
        mesh=mesh,
        scratch_types=(
            [pltpu.VMEM((epw,), jnp.int32)] * 2
            + [pltpu.VMEM((ch,), jnp.int32)] * nb
            + [pltpu.VMEM((ch,), jnp.int32)] * nb
            + [pltpu.VMEM((ch, d), jnp.float32)] * nb
            + [pltpu.VMEM_SHARED((n_pad, d), jnp.float32)]
            + [pltpu.SemaphoreType.DMA] * nb
        ),
    )
    def kern(table_hbm, col_hbm, row_hbm, out_hbm, col_all, row_all, *rest):
        col_v = rest[0:nb]
        row_v = rest[nb:2 * nb]
        rows_v = rest[2 * nb:3 * nb]
        acc_sh = rest[3 * nb]
        gsem = rest[3 * nb + 1:3 * nb + 1 + nb]

        cid = lax.axis_index("c")
        sid = lax.axis_index("s")
        wid = sid * nc + cid

        _memset_rows(rows_v[0], ch, d, 0.0)
        base_row = sid * rpt
        _zero_stripe(acc_sh, rows_v[0], base_row, rpt, ch)
        ebase = wid * epw
        pltpu.sync_copy(col_hbm.at[pl.ds(ebase, epw)], col_all)
        pltpu.sync_copy(row_hbm.at[pl.ds(ebase, epw)], row_all)
        plsc.subcore_barrier()

        def gather(i, b):
            _copy_idx(col_all, col_v[b], i * ch, ch)
            pltpu.async_copy(table_hbm.at[col_v[b]], rows_v[b], gsem[b])

        def wait_gather(b):
            pltpu.make_async_copy(table_hbm.at[pl.ds(0, ch)], rows_v[b],
                                  gsem[b]).wait()

        def scatter(i, b):
            _copy_idx(row_all, row_v[b], i * ch, ch)
            pltpu.sync_copy(rows_v[b], acc_sh.at[row_v[b]], add=True)

        if nb > 1:
            for b in range(nb):
                gather(b, b)

            def round_(r, _):
                i0 = r * nb
                for b in range(nb):
                    wait_gather(b)
                    scatter(i0 + b, b)
                    gather(i0 + nb + b, b)
                return 0

            lax.fori_loop(0, nchunk // nb - 1, round_, 0)
            i0 = nchunk - nb
            for b in range(nb):
                wait_gather(b)
                scatter(i0 + b, b)
        else:
            def step_seq(i, _):
                gather(i, 0)
                wait_gather(0)
                scatter(i, 0)
                return 0

            lax.fori_loop(0, nchunk, step_seq, 0)
        plsc.subcore_barrier()

        pltpu.sync_copy(acc_sh.at[pl.ds(base_row, rpt)],
                        out_hbm.at[cid, pl.ds(base_row, rpt)])

    return kern


_BLK = 1000


def _dvec(deg_ref):
    rs = deg_ref[:, 0:1] + deg_ref[:, 1:2]
    return jnp.where(rs > 0, lax.rsqrt(rs), 0.0)


def _scale_matmul_body(deg_ref, x_ref, w_ref, o_ref):
    o_ref[...] = _dvec(deg_ref) * jnp.dot(
        x_ref[...], w_ref[...], preferred_element_type=jnp.float32)


def _tc_scale_matmul(deg, x, w):
    n, f = x.shape
    h = w.shape[1]
    grid = n // _BLK
    return pl.pallas_call(
        _scale_matmul_body,
        grid=(grid,),
        in_specs=[
            pl.BlockSpec((_BLK, 2), lambda i: (i, 0)),
            pl.BlockSpec((_BLK, f), lambda i: (i, 0)),
            pl.BlockSpec((f, h), lambda i: (0, 0)),
        ],
        out_specs=pl.BlockSpec((_BLK, h), lambda i: (i, 0)),
        out_shape=jax.ShapeDtypeStruct((n, h), jnp.float32),
    )(deg, x, w)


def _relu_matmul_body(deg_ref, a_ref, w_ref, o_ref):
    dv = _dvec(deg_ref)
    hidden = jnp.maximum(dv * (a_ref[0] + a_ref[1]), 0.0)
    o_ref[...] = dv * jnp.dot(
        hidden, w_ref[...], preferred_element_type=jnp.float32)


def _tc_relu_matmul(deg, agg, w, n):
    f = agg.shape[2]
    h = w.shape[1]
    grid = n // _BLK
    return pl.pallas_call(
        _relu_matmul_body,
        grid=(grid,),
        in_specs=[
            pl.BlockSpec((_BLK, 2), lambda i: (i, 0)),
            pl.BlockSpec((2, _BLK, f), lambda i: (0, i, 0)),
            pl.BlockSpec((f, h), lambda i: (0, 0)),
        ],
        out_specs=pl.BlockSpec((_BLK, h), lambda i: (i, 0)),
        out_shape=jax.ShapeDtypeStruct((n, h), jnp.float32),
    )(deg, agg, w)


def _encode_body(c, deg_ref, a_ref, o_ref):
    z = _dvec(deg_ref) * (a_ref[0] + a_ref[1])
    o_ref[...] = z[:, :c]


def _tc_encode(deg, agg, n, c):
    cp = agg.shape[2]
    grid = n // _BLK
    return pl.pallas_call(
        functools.partial(_encode_body, c),
        grid=(grid,),
        in_specs=[
            pl.BlockSpec((_BLK, 2), lambda i: (i, 0)),
            pl.BlockSpec((2, _BLK, cp), lambda i: (0, i, 0)),
        ],
        out_specs=pl.BlockSpec((_BLK, c), lambda i: (i, 0)),
        out_shape=jax.ShapeDtypeStruct((n, c), jnp.float32),
    )(deg, agg)


def _decode_body(zi_ref, zj_ref, o_ref):
    zz = lax.dot_general(
        zi_ref[...], zj_ref[...], (((1,), (1,)), ((), ())),
        preferred_element_type=jnp.float32)
    o_ref[...] = 1.0 / (1.0 + jnp.exp(-zz))


def _tc_decode(z):
    n, c = z.shape
    blk = 2048
    grid = pl.cdiv(n, blk)
    return pl.pallas_call(
        _decode_body,
        grid=(grid, grid),
        in_specs=[
            pl.BlockSpec((blk, c), lambda i, j: (i, 0)),
            pl.BlockSpec((blk, c), lambda i, j: (j, 0)),
        ],
        out_specs=pl.BlockSpec((blk, blk), lambda i, j: (i, j)),
        out_shape=jax.ShapeDtypeStruct((n, n), jnp.float32),
    )(z, z)


def kernel(x, edge_index, W1, W2):
    n, _ = x.shape
    e = edge_index.shape[1]
    row = edge_index[0]
    col = edge_index[1]
    code = W2.shape[1]
    if W2.shape[1] % 128:
        W2 = jnp.pad(W2, ((0, 0), (0, 128 - W2.shape[1] % 128)))
    info = plsc.get_sparse_core_info()
    nw = info.num_cores * info.num_subcores
    e_pad = pl.cdiv(e, nw * 128) * nw * 128
    if e_pad != e:
        row = jnp.pad(row, (0, e_pad - e), constant_values=n)
        col = jnp.pad(col, (0, e_pad - e))
        e = e_pad

    deg = _make_degree(n, e)(row).T
    xf1 = _tc_scale_matmul(deg, x, W1)
    agg1 = _make_edge_aggregate(n, e, W1.shape[1])(xf1, col, row)
    xf2 = _tc_relu_matmul(deg, agg1, W2, n)
    agg2 = _make_edge_aggregate(n, e, W2.shape[1])(xf2, col, row)
    z = _tc_encode(deg, agg2, n, code)
    return _tc_decode(z)


# speedup vs baseline: 1.0569x; 1.0569x over previous
import functools

import jax
import jax.numpy as jnp
from jax import lax
from jax.experimental import pallas as pl
from jax.experimental.pallas import tpu as pltpu
from jax.experimental.pallas import tpu_sc as plsc


def _pick_chunk(epw):
    for ch in (128, 112, 96, 80, 64, 48, 32, 16):
        if epw % ch == 0:
            return ch
    raise ValueError(f"edges per worker {epw} not divisible by 16")


def _memset_rows(ref, nrows, ncols, value):
    vec = jnp.full((16,), value, jnp.float32)

    def body(i, _):
        for j in range(ncols // 16):
            ref[i, pl.ds(j * 16, 16)] = vec
        return 0

    lax.fori_loop(0, nrows, body, 0)


def _memset_vec(ref, length, value):
    vec = jnp.full((16,), value, jnp.float32)
    for k in range(length // 16):
        ref[pl.ds(k * 16, 16)] = vec


def _copy_idx(src, dst, base, ch):
    for k in range(ch // 16):
        dst[pl.ds(k * 16, 16)] = src[pl.ds(base + k * 16, 16)]


def _zero_stripe(acc_sh, zeros_v, base_row, nrows, ch):
    nfull = nrows // ch
    rem = nrows - nfull * ch

    def body(i, _):
        pltpu.sync_copy(zeros_v, acc_sh.at[pl.ds(base_row + i * ch, ch)])
        return 0

    lax.fori_loop(0, nfull, body, 0)
    if rem:
        pltpu.sync_copy(zeros_v.at[pl.ds(0, rem)],
                        acc_sh.at[pl.ds(base_row + nfull * ch, rem)])


@functools.lru_cache(maxsize=None)
def _make_degree(n, e):
    info = plsc.get_sparse_core_info()
    nc, ns = info.num_cores, info.num_subcores
    nw = nc * ns
    epw = e // nw
    ch = _pick_chunk(epw)
    nchunk = epw // ch
    n_pad = pl.cdiv(n + 1, ns * 128) * ns * 128
    rpt = n_pad // ns
    mesh = plsc.VectorSubcoreMesh(core_axis_name="c", subcore_axis_name="s")

    @functools.partial(
        pl.kernel,
        out_type=jax.ShapeDtypeStruct((nc, n_pad), jnp.float32),
        mesh=mesh,
        scratch_types=[
            pltpu.VMEM((epw,), jnp.int32),
            pltpu.VMEM((ch,), jnp.int32),
            pltpu.VMEM((ch,), jnp.float32),
            pltpu.VMEM((ch,), jnp.float32),
            pltpu.VMEM_SHARED((n_pad,), jnp.float32),
        ],
    )
    def kern(row_hbm, out_hbm, row_all, row_v, ones_v, zeros_v, acc_sh):
        cid = lax.axis_index("c")
        sid = lax.axis_index("s")
        wid = sid * nc + cid

        _memset_vec(ones_v, ch, 1.0)
        _memset_vec(zeros_v, ch, 0.0)
        base_row = sid * rpt

        def zbody(i, _):
            pltpu.sync_copy(zeros_v, acc_sh.at[pl.ds(base_row + i * ch, ch)])
            return 0

        lax.fori_loop(0, rpt // ch, zbody, 0)
        if rpt % ch:
            pltpu.sync_copy(zeros_v.at[pl.ds(0, rpt % ch)],
                            acc_sh.at[pl.ds(base_row + (rpt // ch) * ch,
                                            rpt % ch)])
        pltpu.sync_copy(row_hbm.at[pl.ds(wid * epw, epw)], row_all)
        plsc.subcore_barrier()

        def chunk(i, _):
            _copy_idx(row_all, row_v, i * ch, ch)
            pltpu.sync_copy(ones_v, acc_sh.at[row_v], add=True)
            return 0

        lax.fori_loop(0, nchunk, chunk, 0)
        plsc.subcore_barrier()

        pltpu.sync_copy(acc_sh.at[pl.ds(base_row, rpt)],
                        out_hbm.at[cid, pl.ds(base_row, rpt)])

    return kern


_NBUF = 5


@functools.lru_cache(maxsize=None)
def _make_edge_aggregate(n, e, d):
    info = plsc.get_sparse_core_info()
    nc, ns = info.num_cores, info.num_subcores
    nw = nc * ns
    epw = e // nw
    n_pad = pl.cdiv(n + 1, ns * 8) * ns * 8
    rpt = n_pad // ns
    tile_budget = (2097151 - n_pad * d) // ns - 2048
    ch, nb = None, 1
    for c_ in (128, 112, 96, 80, 64, 48, 32, 16):
        if epw % c_:
            continue
        for nb_ in (5, 4, 2):
            if (epw // c_) % nb_:
                continue
            if 2 * epw + 2 * nb_ * c_ + nb_ * c_ * d <= tile_budget:
                break
        else:
            continue
        ch, nb = c_, nb_
        break
    if ch is None:
        ch, nb = _pick_chunk(epw), 1
    nchunk = epw // ch
    mesh = plsc.VectorSubcoreMesh(core_axis_name="c", subcore_axis_name="s")

    @functools.partial(
        pl.kernel,
        out_type=jax.ShapeDtypeStruct((nc, n_pad, d), jnp.float32),
        mesh=mesh,
        scratch_types=(
            [pltpu.VMEM((epw,), jnp.int32)] * 2
            + [pltpu.VMEM((ch,), jnp.int32)] * nb
            + [pltpu.VMEM((ch,), jnp.int32)] * nb
            + [pltpu.VMEM((ch, d), jnp.float32)] * nb
            + [pltpu.VMEM_SHARED((n_pad, d), jnp.float32)]
            + [pltpu.SemaphoreType.DMA] * (2 * nb)
        ),
    )
    def kern(table_hbm, col_hbm, row_hbm, out_hbm, col_all, row_all, *rest):
        col_v = rest[0:nb]
        row_v = rest[nb:2 * nb]
        rows_v = rest[2 * nb:3 * nb]
        acc_sh = rest[3 * nb]
        gsem = rest[3 * nb + 1:3 * nb + 1 + nb]
        ssem = rest[3 * nb + 1 + nb:3 * nb + 1 + 2 * nb]

        cid = lax.axis_index("c")
        sid = lax.axis_index("s")
        wid = sid * nc + cid

        _memset_rows(rows_v[0], ch, d, 0.0)
        base_row = sid * rpt
        _zero_stripe(acc_sh, rows_v[0], base_row, rpt, ch)
        ebase = wid * epw
        pltpu.sync_copy(col_hbm.at[pl.ds(ebase, epw)], col_all)
        pltpu.sync_copy(row_hbm.at[pl.ds(ebase, epw)], row_all)
        plsc.subcore_barrier()

        def gather(i, b):
            _copy_idx(col_all, col_v[b], i * ch, ch)
            pltpu.async_copy(table_hbm.at[col_v[b]], rows_v[b], gsem[b])

        def wait_gather(b):
            pltpu.make_async_copy(table_hbm.at[pl.ds(0, ch)], rows_v[b],
                                  gsem[b]).wait()

        def scatter_async(i, b):
            _copy_idx(row_all, row_v[b], i * ch, ch)
            return pltpu.async_copy(rows_v[b], acc_sh.at[row_v[b]], ssem[b],
                                    add=True)

        if nb > 1:
            for b in range(nb):
                gather(b, b)

            def round_(r, _):
                i0 = r * nb
                descs = []
                for b in range(nb):
                    wait_gather(b)
                    descs.append(scatter_async(i0 + b, b))
                for b in range(nb):
                    descs[b].wait()
                    gather(i0 + nb + b, b)
                return 0

            lax.fori_loop(0, nchunk // nb - 1, round_, 0)
            i0 = nchunk - nb
            descs = []
            for b in range(nb):
                wait_gather(b)
                descs.append(scatter_async(i0 + b, b))
            for b in range(nb):
                descs[b].wait()
        else:
            def step_seq(i, _):
                gather(i, 0)
                wait_gather(0)
                scatter_async(i, 0).wait()
                return 0

            lax.fori_loop(0, nchunk, step_seq, 0)
        plsc.subcore_barrier()

        pltpu.sync_copy(acc_sh.at[pl.ds(base_row, rpt)],
                        out_hbm.at[cid, pl.ds(base_row, rpt)])

    return kern


_BLK = 1000


def _dvec(deg_ref):
    rs = deg_ref[:, 0:1] + deg_ref[:, 1:2]
    return jnp.where(rs > 0, lax.rsqrt(rs), 0.0)


def _scale_matmul_body(deg_ref, x_ref, w_ref, o_ref):
    o_ref[...] = _dvec(deg_ref) * jnp.dot(
        x_ref[...], w_ref[...], preferred_element_type=jnp.float32)


def _tc_scale_matmul(deg, x, w):
    n, f = x.shape
    h = w.shape[1]
    grid = n // _BLK
    return pl.pallas_call(
        _scale_matmul_body,
        grid=(grid,),
        in_specs=[
            pl.BlockSpec((_BLK, 2), lambda i: (i, 0)),
            pl.BlockSpec((_BLK, f), lambda i: (i, 0)),
            pl.BlockSpec((f, h), lambda i: (0, 0)),
        ],
        out_specs=pl.BlockSpec((_BLK, h), lambda i: (i, 0)),
        out_shape=jax.ShapeDtypeStruct((n, h), jnp.float32),
    )(deg, x, w)


def _relu_matmul_body(deg_ref, a_ref, w_ref, o_ref):
    dv = _dvec(deg_ref)
    hidden = jnp.maximum(dv * (a_ref[0] + a_ref[1]), 0.0)
    o_ref[...] = dv * jnp.dot(
        hidden, w_ref[...], preferred_element_type=jnp.float32)


def _tc_relu_matmul(deg, agg, w, n):
    f = agg.shape[2]
    h = w.shape[1]
    grid = n // _BLK
    return pl.pallas_call(
        _relu_matmul_body,
        grid=(grid,),
        in_specs=[
            pl.BlockSpec((_BLK, 2), lambda i: (i, 0)),
            pl.BlockSpec((2, _BLK, f), lambda i: (0, i, 0)),
            pl.BlockSpec((f, h), lambda i: (0, 0)),
        ],
        out_specs=pl.BlockSpec((_BLK, h), lambda i: (i, 0)),
        out_shape=jax.ShapeDtypeStruct((n, h), jnp.float32),
    )(deg, agg, w)


def _encode_body(c, deg_ref, a_ref, o_ref):
    z = _dvec(deg_ref) * (a_ref[0] + a_ref[1])
    o_ref[...] = z[:, :c]


def _tc_encode(deg, agg, n, c):
    cp = agg.shape[2]
    grid = n // _BLK
    return pl.pallas_call(
        functools.partial(_encode_body, c),
        grid=(grid,),
        in_specs=[
            pl.BlockSpec((_BLK, 2), lambda i: (i, 0)),
            pl.BlockSpec((2, _BLK, cp), lambda i: (0, i, 0)),
        ],
        out_specs=pl.BlockSpec((_BLK, c), lambda i: (i, 0)),
        out_shape=jax.ShapeDtypeStruct((n, c), jnp.float32),
    )(deg, agg)


def _decode_body(zi_ref, zj_ref, o_ref):
    zz = lax.dot_general(
        zi_ref[...], zj_ref[...], (((1,), (1,)), ((), ())),
        preferred_element_type=jnp.float32)
    o_ref[...] = 1.0 / (1.0 + jnp.exp(-zz))


def _tc_decode(z):
    n, c = z.shape
    blk = 2048
    grid = pl.cdiv(n, blk)
    return pl.pallas_call(
        _decode_body,
        grid=(grid, grid),
        in_specs=[
            pl.BlockSpec((blk, c), lambda i, j: (i, 0)),
            pl.BlockSpec((blk, c), lambda i, j: (j, 0)),
        ],
        out_specs=pl.BlockSpec((blk, blk), lambda i, j: (i, j)),
        out_shape=jax.ShapeDtypeStruct((n, n), jnp.float32),
    )(z, z)


def kernel(x, edge_index, W1, W2):
    n, _ = x.shape
    e = edge_index.shape[1]
    row = edge_index[0]
    col = edge_index[1]
    code = W2.shape[1]
    if W2.shape[1] % 128:
        W2 = jnp.pad(W2, ((0, 0), (0, 128 - W2.shape[1] % 128)))
    info = plsc.get_sparse_core_info()
    nw = info.num_cores * info.num_subcores
    e_pad = pl.cdiv(e, nw * 128) * nw * 128
    if e_pad != e:
        row = jnp.pad(row, (0, e_pad - e), constant_values=n)
        col = jnp.pad(col, (0, e_pad - e))
        e = e_pad

    deg = _make_degree(n, e)(row).T
    xf1 = _tc_scale_matmul(deg, x, W1)
    agg1 = _make_edge_aggregate(n, e, W1.shape[1])(xf1, col, row)
    xf2 = _tc_relu_matmul(deg, agg1, W2, n)
    agg2 = _make_edge_aggregate(n, e, W2.shape[1])(xf2, col, row)
    z = _tc_encode(deg, agg2, n, code)
    return _tc_decode(z)


# speedup vs baseline: 1.1389x; 1.0776x over previous
import functools

import jax
import jax.numpy as jnp
from jax import lax
from jax.experimental import pallas as pl
from jax.experimental.pallas import tpu as pltpu
from jax.experimental.pallas import tpu_sc as plsc


def _pick_chunk(epw):
    for ch in (128, 112, 96, 80, 64, 48, 32, 16):
        if epw % ch == 0:
            return ch
    raise ValueError(f"edges per worker {epw} not divisible by 16")


def _memset_rows(ref, nrows, ncols, value):
    vec = jnp.full((16,), value, jnp.float32)

    def body(i, _):
        for j in range(ncols // 16):
            ref[i, pl.ds(j * 16, 16)] = vec
        return 0

    lax.fori_loop(0, nrows, body, 0)


def _memset_vec(ref, length, value):
    vec = jnp.full((16,), value, jnp.float32)
    for k in range(length // 16):
        ref[pl.ds(k * 16, 16)] = vec


def _copy_idx(src, dst, base, ch):
    for k in range(ch // 16):
        dst[pl.ds(k * 16, 16)] = src[pl.ds(base + k * 16, 16)]


def _zero_stripe(acc_sh, zeros_v, base_row, nrows, ch):
    nfull = nrows // ch
    rem = nrows - nfull * ch

    def body(i, _):
        pltpu.sync_copy(zeros_v, acc_sh.at[pl.ds(base_row + i * ch, ch)])
        return 0

    lax.fori_loop(0, nfull, body, 0)
    if rem:
        pltpu.sync_copy(zeros_v.at[pl.ds(0, rem)],
                        acc_sh.at[pl.ds(base_row + nfull * ch, rem)])


@functools.lru_cache(maxsize=None)
def _make_degree(n, e):
    info = plsc.get_sparse_core_info()
    nc, ns = info.num_cores, info.num_subcores
    nw = nc * ns
    epw = e // nw
    ch = _pick_chunk(epw)
    nchunk = epw // ch
    n_pad = pl.cdiv(n + 64, ns * 128) * ns * 128
    rpt = n_pad // ns
    mesh = plsc.VectorSubcoreMesh(core_axis_name="c", subcore_axis_name="s")

    @functools.partial(
        pl.kernel,
        out_type=jax.ShapeDtypeStruct((nc, n_pad), jnp.float32),
        mesh=mesh,
        scratch_types=[
            pltpu.VMEM((epw,), jnp.int32),
            pltpu.VMEM((ch,), jnp.int32),
            pltpu.VMEM((ch,), jnp.float32),
            pltpu.VMEM((ch,), jnp.float32),
            pltpu.VMEM_SHARED((n_pad,), jnp.float32),
        ],
    )
    def kern(row_hbm, out_hbm, row_all, row_v, ones_v, zeros_v, acc_sh):
        cid = lax.axis_index("c")
        sid = lax.axis_index("s")
        wid = sid * nc + cid

        _memset_vec(ones_v, ch, 1.0)
        _memset_vec(zeros_v, ch, 0.0)
        base_row = sid * rpt

        def zbody(i, _):
            pltpu.sync_copy(zeros_v, acc_sh.at[pl.ds(base_row + i * ch, ch)])
            return 0

        lax.fori_loop(0, rpt // ch, zbody, 0)
        if rpt % ch:
            pltpu.sync_copy(zeros_v.at[pl.ds(0, rpt % ch)],
                            acc_sh.at[pl.ds(base_row + (rpt // ch) * ch,
                                            rpt % ch)])
        pltpu.sync_copy(row_hbm.at[pl.ds(wid * epw, epw)], row_all)
        plsc.subcore_barrier()

        def chunk(i, _):
            _copy_idx(row_all, row_v, i * ch, ch)
            pltpu.sync_copy(ones_v, acc_sh.at[row_v], add=True)
            return 0

        lax.fori_loop(0, nchunk, chunk, 0)
        plsc.subcore_barrier()

        pltpu.sync_copy(acc_sh.at[pl.ds(base_row, rpt)],
                        out_hbm.at[cid, pl.ds(base_row, rpt)])

    return kern


_NBUF = 5


@functools.lru_cache(maxsize=None)
def _make_edge_aggregate(n, e, d):
    info = plsc.get_sparse_core_info()
    nc, ns = info.num_cores, info.num_subcores
    nw = nc * ns
    epw = e // nw
    n_pad = pl.cdiv(n + 64, ns * 8) * ns * 8
    rpt = n_pad // ns
    tile_budget = (2097151 - n_pad * d) // ns - 2048
    ch, nb = None, 1
    for c_ in (128, 112, 96, 80, 64, 48, 32, 16):
        if epw % c_:
            continue
        if 2 * epw + 4 * c_ + 2 * c_ * d <= tile_budget:
            ch, nb = c_, 2
            break
    if ch is None:
        ch, nb = _pick_chunk(epw), 1
    nchunk = epw // ch
    mesh = plsc.VectorSubcoreMesh(core_axis_name="c", subcore_axis_name="s")

    @functools.partial(
        pl.kernel,
        out_type=jax.ShapeDtypeStruct((nc, n_pad, d), jnp.float32),
        mesh=mesh,
        scratch_types=(
            [pltpu.VMEM((epw,), jnp.int32)] * 2
            + [pltpu.VMEM((ch,), jnp.int32)] * nb
            + [pltpu.VMEM((ch,), jnp.int32)] * nb
            + [pltpu.VMEM((ch, d), jnp.float32)] * nb
            + [pltpu.VMEM_SHARED((n_pad, d), jnp.float32)]
            + [pltpu.SemaphoreType.DMA] * nb
        ),
    )
    def kern(table_hbm, col_hbm, row_hbm, out_hbm, col_all, row_all, *rest):
        col_v = rest[0:nb]
        row_v = rest[nb:2 * nb]
        rows_v = rest[2 * nb:3 * nb]
        acc_sh = rest[3 * nb]
        gsem = rest[3 * nb + 1:3 * nb + 1 + nb]

        cid = lax.axis_index("c")
        sid = lax.axis_index("s")
        wid = sid * nc + cid

        _memset_rows(rows_v[0], ch, d, 0.0)
        base_row = sid * rpt
        _zero_stripe(acc_sh, rows_v[0], base_row, rpt, ch)
        ebase = wid * epw
        pltpu.sync_copy(col_hbm.at[pl.ds(ebase, epw)], col_all)
        pltpu.sync_copy(row_hbm.at[pl.ds(ebase, epw)], row_all)
        plsc.subcore_barrier()

        def gather(i, b):
            _copy_idx(col_all, col_v[b], i * ch, ch)
            pltpu.async_copy(table_hbm.at[col_v[b]], rows_v[b], gsem[b])

        def wait_gather(b):
            pltpu.make_async_copy(table_hbm.at[pl.ds(0, ch)], rows_v[b],
                                  gsem[b]).wait()

        def scatter(i, b):
            _copy_idx(row_all, row_v[b], i * ch, ch)
            pltpu.sync_copy(rows_v[b], acc_sh.at[row_v[b]], add=True)

        if nb > 1 and nchunk >= 4:
            gather(0, 0)
            odd = nchunk % 2
            rounds = (nchunk - 1) // 2 if odd else nchunk // 2 - 1

            def step(g, _):
                i0 = g * 2
                gather(i0 + 1, 1)
                wait_gather(0)
                scatter(i0, 0)
                gather(i0 + 2, 0)
                wait_gather(1)
                scatter(i0 + 1, 1)
                return 0

            lax.fori_loop(0, rounds, step, 0)
            if odd:
                wait_gather(0)
                scatter(nchunk - 1, 0)
            else:
                gather(nchunk - 1, 1)
                wait_gather(0)
                scatter(nchunk - 2, 0)
                wait_gather(1)
                scatter(nchunk - 1, 1)
        else:
            def step_seq(i, _):
                gather(i, 0)
                wait_gather(0)
                scatter(i, 0)
                return 0

            lax.fori_loop(0, nchunk, step_seq, 0)
        plsc.subcore_barrier()

        pltpu.sync_copy(acc_sh.at[pl.ds(base_row, rpt)],
                        out_hbm.at[cid, pl.ds(base_row, rpt)])

    return kern


_BLK = 1000


def _dvec(deg_ref):
    rs = deg_ref[:, 0:1] + deg_ref[:, 1:2]
    return jnp.where(rs > 0, lax.rsqrt(rs), 0.0)


def _scale_matmul_body(deg_ref, x_ref, w_ref, o_ref):
    o_ref[...] = _dvec(deg_ref) * jnp.dot(
        x_ref[...], w_ref[...], preferred_element_type=jnp.float32)


def _tc_scale_matmul(deg, x, w):
    n, f = x.shape
    h = w.shape[1]
    grid = n // _BLK
    return pl.pallas_call(
        _scale_matmul_body,
        grid=(grid,),
        in_specs=[
            pl.BlockSpec((_BLK, 2), lambda i: (i, 0)),
            pl.BlockSpec((_BLK, f), lambda i: (i, 0)),
            pl.BlockSpec((f, h), lambda i: (0, 0)),
        ],
        out_specs=pl.BlockSpec((_BLK, h), lambda i: (i, 0)),
        out_shape=jax.ShapeDtypeStruct((n, h), jnp.float32),
    )(deg, x, w)


def _relu_matmul_body(deg_ref, a_ref, w_ref, o_ref):
    dv = _dvec(deg_ref)
    hidden = jnp.maximum(dv * (a_ref[0] + a_ref[1]), 0.0)
    o_ref[...] = dv * jnp.dot(
        hidden, w_ref[...], preferred_element_type=jnp.float32)


def _tc_relu_matmul(deg, agg, w, n):
    f = agg.shape[2]
    h = w.shape[1]
    grid = n // _BLK
    return pl.pallas_call(
        _relu_matmul_body,
        grid=(grid,),
        in_specs=[
            pl.BlockSpec((_BLK, 2), lambda i: (i, 0)),
            pl.BlockSpec((2, _BLK, f), lambda i: (0, i, 0)),
            pl.BlockSpec((f, h), lambda i: (0, 0)),
        ],
        out_specs=pl.BlockSpec((_BLK, h), lambda i: (i, 0)),
        out_shape=jax.ShapeDtypeStruct((n, h), jnp.float32),
    )(deg, agg, w)


def _encode_body(c, deg_ref, a_ref, o_ref):
    z = _dvec(deg_ref) * (a_ref[0] + a_ref[1])
    o_ref[...] = z[:, :c]


def _tc_encode(deg, agg, n, c):
    cp = agg.shape[2]
    grid = n // _BLK
    return pl.pallas_call(
        functools.partial(_encode_body, c),
        grid=(grid,),
        in_specs=[
            pl.BlockSpec((_BLK, 2), lambda i: (i, 0)),
            pl.BlockSpec((2, _BLK, cp), lambda i: (0, i, 0)),
        ],
        out_specs=pl.BlockSpec((_BLK, c), lambda i: (i, 0)),
        out_shape=jax.ShapeDtypeStruct((n, c), jnp.float32),
    )(deg, agg)


def _decode_body(zi_ref, zj_ref, o_ref):
    zz = lax.dot_general(
        zi_ref[...], zj_ref[...], (((1,), (1,)), ((), ())),
        preferred_element_type=jnp.float32)
    o_ref[...] = 1.0 / (1.0 + jnp.exp(-zz))


def _tc_decode(z):
    n, c = z.shape
    blk = 2048
    grid = pl.cdiv(n, blk)
    return pl.pallas_call(
        _decode_body,
        grid=(grid, grid),
        in_specs=[
            pl.BlockSpec((blk, c), lambda i, j: (i, 0)),
            pl.BlockSpec((blk, c), lambda i, j: (j, 0)),
        ],
        out_specs=pl.BlockSpec((blk, blk), lambda i, j: (i, j)),
        out_shape=jax.ShapeDtypeStruct((n, n), jnp.float32),
    )(z, z)


def kernel(x, edge_index, W1, W2):
    n, _ = x.shape
    e = edge_index.shape[1]
    row = edge_index[0]
    col = edge_index[1]
    code = W2.shape[1]
    if W2.shape[1] % 128:
        W2 = jnp.pad(W2, ((0, 0), (0, 128 - W2.shape[1] % 128)))
    info = plsc.get_sparse_core_info()
    nw = info.num_cores * info.num_subcores
    e_pad = pl.cdiv(e, nw * 128) * nw * 128
    if e_pad != e:
        spread = n + (jnp.arange(e_pad - e, dtype=jnp.int32) % 64)
        row = jnp.concatenate([row, spread])
        col = jnp.pad(col, (0, e_pad - e))
        e = e_pad

    deg = _make_degree(n, e)(row).T
    xf1 = _tc_scale_matmul(deg, x, W1)
    agg1 = _make_edge_aggregate(n, e, W1.shape[1])(xf1, col, row)
    xf2 = _tc_relu_matmul(deg, agg1, W2, n)
    agg2 = _make_edge_aggregate(n, e, W2.shape[1])(xf2, col, row)
    z = _tc_encode(deg, agg2, n, code)
    return _tc_decode(z)


# speedup vs baseline: 2.0140x; 1.7683x over previous
import functools

import jax
import jax.numpy as jnp
from jax import lax
from jax.experimental import pallas as pl
from jax.experimental.pallas import tpu as pltpu
from jax.experimental.pallas import tpu_sc as plsc


def _pick_chunk(epw):
    for ch in (128, 112, 96, 80, 64, 48, 32, 16):
        if epw % ch == 0:
            return ch
    raise ValueError(f"edges per worker {epw} not divisible by 16")


def _memset_rows(ref, nrows, ncols, value):
    vec = jnp.full((16,), value, jnp.float32)

    def body(i, _):
        for j in range(ncols // 16):
            ref[i, pl.ds(j * 16, 16)] = vec
        return 0

    lax.fori_loop(0, nrows, body, 0)


def _memset_vec(ref, length, value):
    vec = jnp.full((16,), value, jnp.float32)
    for k in range(length // 16):
        ref[pl.ds(k * 16, 16)] = vec


def _copy_idx(src, dst, base, ch):
    for k in range(ch // 16):
        dst[pl.ds(k * 16, 16)] = src[pl.ds(base + k * 16, 16)]


def _zero_stripe(acc_sh, zeros_v, base_row, nrows, ch):
    nfull = nrows // ch
    rem = nrows - nfull * ch

    def body(i, _):
        pltpu.sync_copy(zeros_v, acc_sh.at[pl.ds(base_row + i * ch, ch)])
        return 0

    lax.fori_loop(0, nfull, body, 0)
    if rem:
        pltpu.sync_copy(zeros_v.at[pl.ds(0, rem)],
                        acc_sh.at[pl.ds(base_row + nfull * ch, rem)])


@functools.lru_cache(maxsize=None)
def _make_degree(n, e):
    info = plsc.get_sparse_core_info()
    nc, ns = info.num_cores, info.num_subcores
    nw = nc * ns
    epw = e // nw
    ch = _pick_chunk(epw)
    nchunk = epw // ch
    n_pad = pl.cdiv(n + 64, ns * 128) * ns * 128
    rpt = n_pad // ns
    mesh = plsc.VectorSubcoreMesh(core_axis_name="c", subcore_axis_name="s")

    @functools.partial(
        pl.kernel,
        out_type=jax.ShapeDtypeStruct((nc, n_pad), jnp.float32),
        mesh=mesh,
        scratch_types=[
            pltpu.VMEM((epw,), jnp.int32),
            pltpu.VMEM((ch,), jnp.int32),
            pltpu.VMEM((ch,), jnp.float32),
            pltpu.VMEM((ch,), jnp.float32),
            pltpu.VMEM_SHARED((n_pad,), jnp.float32),
        ],
    )
    def kern(row_hbm, out_hbm, row_all, row_v, ones_v, zeros_v, acc_sh):
        cid = lax.axis_index("c")
        sid = lax.axis_index("s")
        wid = sid * nc + cid

        _memset_vec(ones_v, ch, 1.0)
        _memset_vec(zeros_v, ch, 0.0)
        base_row = sid * rpt

        def zbody(i, _):
            pltpu.sync_copy(zeros_v, acc_sh.at[pl.ds(base_row + i * ch, ch)])
            return 0

        lax.fori_loop(0, rpt // ch, zbody, 0)
        if rpt % ch:
            pltpu.sync_copy(zeros_v.at[pl.ds(0, rpt % ch)],
                            acc_sh.at[pl.ds(base_row + (rpt // ch) * ch,
                                            rpt % ch)])
        pltpu.sync_copy(row_hbm.at[pl.ds(wid * epw, epw)], row_all)
        plsc.subcore_barrier()

        def chunk(i, _):
            _copy_idx(row_all, row_v, i * ch, ch)
            pltpu.sync_copy(ones_v, acc_sh.at[row_v], add=True)
            return 0

        lax.fori_loop(0, nchunk, chunk, 0)
        plsc.subcore_barrier()

        pltpu.sync_copy(acc_sh.at[pl.ds(base_row, rpt)],
                        out_hbm.at[cid, pl.ds(base_row, rpt)])

    return kern


_NBUF = 5


@functools.lru_cache(maxsize=None)
def _make_edge_aggregate(n, e, d):
    info = plsc.get_sparse_core_info()
    nc, ns = info.num_cores, info.num_subcores
    nw = nc * ns
    epw = e // nw
    n_pad = pl.cdiv(n + 64, ns * 8) * ns * 8
    rpt = n_pad // ns
    tile_budget = (2097151 - n_pad * d) // ns - 2048
    ch, nb = None, 1
    for c_ in (128, 112, 96, 80, 64, 48, 32, 16):
        if epw % c_:
            continue
        if 2 * epw + 4 * c_ + 2 * c_ * d <= tile_budget:
            ch, nb = c_, 2
            break
    if ch is None:
        ch, nb = _pick_chunk(epw), 1
    nchunk = epw // ch
    mesh = plsc.VectorSubcoreMesh(core_axis_name="c", subcore_axis_name="s")

    @functools.partial(
        pl.kernel,
        out_type=jax.ShapeDtypeStruct((nc, n_pad, d), jnp.float32),
        mesh=mesh,
        scratch_types=(
            [pltpu.VMEM((epw,), jnp.int32)] * 2
            + [pltpu.VMEM((ch,), jnp.int32)] * nb
            + [pltpu.VMEM((ch,), jnp.int32)] * nb
            + [pltpu.VMEM((ch, d), jnp.float32)] * nb
            + [pltpu.VMEM_SHARED((n_pad, d), jnp.float32)]
            + [pltpu.SemaphoreType.DMA] * nb
        ),
    )
    def kern(table_hbm, col_hbm, row_hbm, out_hbm, col_all, row_all, *rest):
        col_v = rest[0:nb]
        row_v = rest[nb:2 * nb]
        rows_v = rest[2 * nb:3 * nb]
        acc_sh = rest[3 * nb]
        gsem = rest[3 * nb + 1:3 * nb + 1 + nb]

        cid = lax.axis_index("c")
        sid = lax.axis_index("s")
        wid = sid * nc + cid

        _memset_rows(rows_v[0], ch, d, 0.0)
        base_row = sid * rpt
        _zero_stripe(acc_sh, rows_v[0], base_row, rpt, ch)
        ebase = wid * epw
        pltpu.sync_copy(col_hbm.at[pl.ds(ebase, epw)], col_all)
        pltpu.sync_copy(row_hbm.at[pl.ds(ebase, epw)], row_all)
        plsc.subcore_barrier()

        def gather(i, b):
            _copy_idx(col_all, col_v[b], i * ch, ch)
            pltpu.async_copy(table_hbm.at[col_v[b]], rows_v[b], gsem[b])

        def wait_gather(b):
            pltpu.make_async_copy(table_hbm.at[pl.ds(0, ch)], rows_v[b],
                                  gsem[b]).wait()

        def scatter(i, b):
            _copy_idx(row_all, row_v[b], i * ch, ch)
            pltpu.sync_copy(rows_v[b], acc_sh.at[row_v[b]], add=True)

        if nb > 1 and nchunk >= 4:
            gather(0, 0)
            odd = nchunk % 2
            rounds = (nchunk - 1) // 2 if odd else nchunk // 2 - 1

            def step(g, _):
                i0 = g * 2
                gather(i0 + 1, 1)
                wait_gather(0)
                scatter(i0, 0)
                gather(i0 + 2, 0)
                wait_gather(1)
                scatter(i0 + 1, 1)
                return 0

            lax.fori_loop(0, rounds, step, 0)
            if odd:
                wait_gather(0)
                scatter(nchunk - 1, 0)
            else:
                gather(nchunk - 1, 1)
                wait_gather(0)
                scatter(nchunk - 2, 0)
                wait_gather(1)
                scatter(nchunk - 1, 1)
        else:
            def step_seq(i, _):
                gather(i, 0)
                wait_gather(0)
                scatter(i, 0)
                return 0

            lax.fori_loop(0, nchunk, step_seq, 0)
        plsc.subcore_barrier()

        pltpu.sync_copy(acc_sh.at[pl.ds(base_row, rpt)],
                        out_hbm.at[cid, pl.ds(base_row, rpt)])

    return kern


_BLK = 1000


def _dvec(deg_ref):
    rs = deg_ref[:, 0:1] + deg_ref[:, 1:2]
    return jnp.where(rs > 0, lax.rsqrt(rs), 0.0)


def _scale_matmul_body(deg_ref, x_ref, w_ref, o_ref):
    o_ref[...] = _dvec(deg_ref) * jnp.dot(
        x_ref[...], w_ref[...], preferred_element_type=jnp.float32)


def _tc_scale_matmul(deg, x, w):
    n, f = x.shape
    h = w.shape[1]
    grid = n // _BLK
    return pl.pallas_call(
        _scale_matmul_body,
        grid=(grid,),
        in_specs=[
            pl.BlockSpec((_BLK, 2), lambda i: (i, 0)),
            pl.BlockSpec((_BLK, f), lambda i: (i, 0)),
            pl.BlockSpec((f, h), lambda i: (0, 0)),
        ],
        out_specs=pl.BlockSpec((_BLK, h), lambda i: (i, 0)),
        out_shape=jax.ShapeDtypeStruct((n, h), jnp.float32),
    )(deg, x, w)


def _relu_matmul_body(deg_ref, a_ref, w_ref, o_ref):
    dv = _dvec(deg_ref)
    hidden = jnp.maximum(dv * (a_ref[0] + a_ref[1]), 0.0)
    o_ref[...] = dv * jnp.dot(
        hidden, w_ref[...], preferred_element_type=jnp.float32)


def _tc_relu_matmul(deg, agg, w, n):
    f = agg.shape[2]
    h = w.shape[1]
    grid = n // _BLK
    return pl.pallas_call(
        _relu_matmul_body,
        grid=(grid,),
        in_specs=[
            pl.BlockSpec((_BLK, 2), lambda i: (i, 0)),
            pl.BlockSpec((2, _BLK, f), lambda i: (0, i, 0)),
            pl.BlockSpec((f, h), lambda i: (0, 0)),
        ],
        out_specs=pl.BlockSpec((_BLK, h), lambda i: (i, 0)),
        out_shape=jax.ShapeDtypeStruct((n, h), jnp.float32),
    )(deg, agg, w)


def _encode_body(c, deg_ref, a_ref, o_ref):
    z = _dvec(deg_ref) * (a_ref[0] + a_ref[1])
    o_ref[...] = z[:, :c]


def _tc_encode(deg, agg, n, c):
    cp = agg.shape[2]
    grid = n // _BLK
    return pl.pallas_call(
        functools.partial(_encode_body, c),
        grid=(grid,),
        in_specs=[
            pl.BlockSpec((_BLK, 2), lambda i: (i, 0)),
            pl.BlockSpec((2, _BLK, cp), lambda i: (0, i, 0)),
        ],
        out_specs=pl.BlockSpec((_BLK, c), lambda i: (i, 0)),
        out_shape=jax.ShapeDtypeStruct((n, c), jnp.float32),
    )(deg, agg)


def _decode_body(zi_ref, zj_ref, o_ref):
    zz = lax.dot_general(
        zi_ref[...], zj_ref[...], (((1,), (1,)), ((), ())),
        preferred_element_type=jnp.float32)
    o_ref[...] = 1.0 / (1.0 + jnp.exp(-zz))


def _tc_decode(z):
    n, c = z.shape
    blk = 2048
    grid = pl.cdiv(n, blk)
    return pl.pallas_call(
        _decode_body,
        grid=(grid, grid),
        in_specs=[
            pl.BlockSpec((blk, c), lambda i, j: (i, 0)),
            pl.BlockSpec((blk, c), lambda i, j: (j, 0)),
        ],
        out_specs=pl.BlockSpec((blk, blk), lambda i, j: (i, j)),
        out_shape=jax.ShapeDtypeStruct((n, n), jnp.float32),
    )(z, z)


def kernel(x, edge_index, W1, W2):
    n, _ = x.shape
    e = edge_index.shape[1]
    row = edge_index[0]
    col = edge_index[1]
    code = W2.shape[1]
    if W2.shape[1] % 128:
        W2 = jnp.pad(W2, ((0, 0), (0, 128 - W2.shape[1] % 128)))
    info = plsc.get_sparse_core_info()
    nw = info.num_cores * info.num_subcores
    e_pad = pl.cdiv(e, nw * 16) * nw * 16
    if e_pad != e:
        spread = n + (jnp.arange(e_pad - e, dtype=jnp.int32) % 64)
        row = jnp.concatenate([row, spread])
        col = jnp.pad(col, (0, e_pad - e))
        e = e_pad

    deg = _make_degree(n, e)(row).T
    xf1 = _tc_scale_matmul(deg, x, W1)
    agg1 = _make_edge_aggregate(n, e, W1.shape[1])(xf1, col, row)
    xf2 = _tc_relu_matmul(deg, agg1, W2, n)
    agg2 = _make_edge_aggregate(n, e, W2.shape[1])(xf2, col, row)
    z = _tc_encode(deg, agg2, n, code)
    return _tc_decode(z)


# speedup vs baseline: 2.0168x; 1.0014x over previous
import functools

import jax
import jax.numpy as jnp
from jax import lax
from jax.experimental import pallas as pl
from jax.experimental.pallas import tpu as pltpu
from jax.experimental.pallas import tpu_sc as plsc


def _pick_chunk(epw):
    for ch in (128, 112, 96, 80, 64, 48, 32, 16):
        if epw % ch == 0:
            return ch
    raise ValueError(f"edges per worker {epw} not divisible by 16")


def _memset_rows(ref, nrows, ncols, value):
    vec = jnp.full((16,), value, jnp.float32)

    def body(i, _):
        for j in range(ncols // 16):
            ref[i, pl.ds(j * 16, 16)] = vec
        return 0

    lax.fori_loop(0, nrows, body, 0)


def _memset_vec(ref, length, value):
    vec = jnp.full((16,), value, jnp.float32)
    for k in range(length // 16):
        ref[pl.ds(k * 16, 16)] = vec


def _copy_idx(src, dst, base, ch):
    for k in range(ch // 16):
        dst[pl.ds(k * 16, 16)] = src[pl.ds(base + k * 16, 16)]


def _zero_stripe(acc_sh, zeros_v, base_row, nrows, ch):
    nfull = nrows // ch
    rem = nrows - nfull * ch

    def body(i, _):
        pltpu.sync_copy(zeros_v, acc_sh.at[pl.ds(base_row + i * ch, ch)])
        return 0

    lax.fori_loop(0, nfull, body, 0)
    if rem:
        pltpu.sync_copy(zeros_v.at[pl.ds(0, rem)],
                        acc_sh.at[pl.ds(base_row + nfull * ch, rem)])


@functools.lru_cache(maxsize=None)
def _make_degree(n, e):
    info = plsc.get_sparse_core_info()
    nc, ns = info.num_cores, info.num_subcores
    nw = nc * ns
    epw = e // nw
    ch = _pick_chunk(epw)
    nchunk = epw // ch
    n_pad = pl.cdiv(n + 64, ns * 128) * ns * 128
    rpt = n_pad // ns
    mesh = plsc.VectorSubcoreMesh(core_axis_name="c", subcore_axis_name="s")

    @functools.partial(
        pl.kernel,
        out_type=jax.ShapeDtypeStruct((nc, n_pad), jnp.float32),
        mesh=mesh,
        scratch_types=[
            pltpu.VMEM((epw,), jnp.int32),
            pltpu.VMEM((ch,), jnp.int32),
            pltpu.VMEM((ch,), jnp.float32),
            pltpu.VMEM((ch,), jnp.float32),
            pltpu.VMEM_SHARED((n_pad,), jnp.float32),
        ],
    )
    def kern(row_hbm, out_hbm, row_all, row_v, ones_v, zeros_v, acc_sh):
        cid = lax.axis_index("c")
        sid = lax.axis_index("s")
        wid = sid * nc + cid

        _memset_vec(ones_v, ch, 1.0)
        _memset_vec(zeros_v, ch, 0.0)
        base_row = sid * rpt

        def zbody(i, _):
            pltpu.sync_copy(zeros_v, acc_sh.at[pl.ds(base_row + i * ch, ch)])
            return 0

        lax.fori_loop(0, rpt // ch, zbody, 0)
        if rpt % ch:
            pltpu.sync_copy(zeros_v.at[pl.ds(0, rpt % ch)],
                            acc_sh.at[pl.ds(base_row + (rpt // ch) * ch,
                                            rpt % ch)])
        pltpu.sync_copy(row_hbm.at[pl.ds(wid * epw, epw)], row_all)
        plsc.subcore_barrier()

        def chunk(i, _):
            _copy_idx(row_all, row_v, i * ch, ch)
            pltpu.sync_copy(ones_v, acc_sh.at[row_v], add=True)
            return 0

        lax.fori_loop(0, nchunk, chunk, 0)
        plsc.subcore_barrier()

        pltpu.sync_copy(acc_sh.at[pl.ds(base_row, rpt)],
                        out_hbm.at[cid, pl.ds(base_row, rpt)])

    return kern


_NBUF = 5


@functools.lru_cache(maxsize=None)
def _make_edge_aggregate(n, e, d):
    info = plsc.get_sparse_core_info()
    nc, ns = info.num_cores, info.num_subcores
    nw = nc * ns
    epw = e // nw
    n_pad = pl.cdiv(n + 64, ns * 8) * ns * 8
    rpt = n_pad // ns
    tile_budget = (2097151 - n_pad * d) // ns - 2048
    ch, nb = None, 1
    for c_ in (128, 112, 96, 80, 64, 48, 32, 16):
        if epw % c_:
            continue
        if 2 * epw + 4 * c_ + 2 * c_ * d <= tile_budget:
            ch, nb = c_, 2
            break
    if ch is None:
        ch, nb = _pick_chunk(epw), 1
    nchunk = epw // ch
    cha = (ch // 2 + 15) // 16 * 16 if ch > 16 else ch
    chb = ch - cha
    mesh = plsc.VectorSubcoreMesh(core_axis_name="c", subcore_axis_name="s")

    @functools.partial(
        pl.kernel,
        out_type=jax.ShapeDtypeStruct((nc, n_pad, d), jnp.float32),
        mesh=mesh,
        scratch_types=(
            [pltpu.VMEM((epw,), jnp.int32)] * 2
            + [pltpu.VMEM((ch,), jnp.int32)] * nb
            + [pltpu.VMEM((cha,), jnp.int32)] * nb
            + [pltpu.VMEM((max(chb, 16),), jnp.int32)] * nb
            + [pltpu.VMEM((ch, d), jnp.float32)] * nb
            + [pltpu.VMEM_SHARED((n_pad, d), jnp.float32)]
            + [pltpu.SemaphoreType.DMA] * (2 * nb)
        ),
    )
    def kern(table_hbm, col_hbm, row_hbm, out_hbm, col_all, row_all, *rest):
        col_v = rest[0:nb]
        row_va = rest[nb:2 * nb]
        row_vb = rest[2 * nb:3 * nb]
        rows_v = rest[3 * nb:4 * nb]
        acc_sh = rest[4 * nb]
        gsem = rest[4 * nb + 1:4 * nb + 1 + nb]
        ssem = rest[4 * nb + 1 + nb:4 * nb + 1 + 2 * nb]

        cid = lax.axis_index("c")
        sid = lax.axis_index("s")
        wid = sid * nc + cid

        _memset_rows(rows_v[0], ch, d, 0.0)
        base_row = sid * rpt
        _zero_stripe(acc_sh, rows_v[0], base_row, rpt, ch)
        ebase = wid * epw
        pltpu.sync_copy(col_hbm.at[pl.ds(ebase, epw)], col_all)
        pltpu.sync_copy(row_hbm.at[pl.ds(ebase, epw)], row_all)
        plsc.subcore_barrier()

        def gather(i, b):
            _copy_idx(col_all, col_v[b], i * ch, ch)
            pltpu.async_copy(table_hbm.at[col_v[b].at[pl.ds(0, cha)]],
                             rows_v[b].at[pl.ds(0, cha)], gsem[b])
            if chb:
                pltpu.async_copy(table_hbm.at[col_v[b].at[pl.ds(cha, chb)]],
                                 rows_v[b].at[pl.ds(cha, chb)], gsem[b])

        def wait_gather(b):
            pltpu.make_async_copy(table_hbm.at[pl.ds(0, ch)], rows_v[b],
                                  gsem[b]).wait()

        def scatter(i, b):
            _copy_idx(row_all, row_va[b], i * ch, cha)
            if chb:
                _copy_idx(row_all, row_vb[b], i * ch + cha, chb)
            d1 = pltpu.async_copy(rows_v[b].at[pl.ds(0, cha)],
                                  acc_sh.at[row_va[b]], ssem[b], add=True)
            if chb:
                d2 = pltpu.async_copy(rows_v[b].at[pl.ds(cha, chb)],
                                      acc_sh.at[row_vb[b]], ssem[b], add=True)
            d1.wait()
            if chb:
                d2.wait()

        if nb > 1 and nchunk >= 4:
            gather(0, 0)
            odd = nchunk % 2
            rounds = (nchunk - 1) // 2 if odd else nchunk // 2 - 1

            def step(g, _):
                i0 = g * 2
                gather(i0 + 1, 1)
                wait_gather(0)
                scatter(i0, 0)
                gather(i0 + 2, 0)
                wait_gather(1)
                scatter(i0 + 1, 1)
                return 0

            lax.fori_loop(0, rounds, step, 0)
            if odd:
                wait_gather(0)
                scatter(nchunk - 1, 0)
            else:
                gather(nchunk - 1, 1)
                wait_gather(0)
                scatter(nchunk - 2, 0)
                wait_gather(1)
                scatter(nchunk - 1, 1)
        else:
            def step_seq(i, _):
                gather(i, 0)
                wait_gather(0)
                scatter(i, 0)
                return 0

            lax.fori_loop(0, nchunk, step_seq, 0)
        plsc.subcore_barrier()

        pltpu.sync_copy(acc_sh.at[pl.ds(base_row, rpt)],
                        out_hbm.at[cid, pl.ds(base_row, rpt)])

    return kern


_BLK = 1000


def _dvec(deg_ref):
    rs = deg_ref[:, 0:1] + deg_ref[:, 1:2]
    return jnp.where(rs > 0, lax.rsqrt(rs), 0.0)


def _scale_matmul_body(deg_ref, x_ref, w_ref, o_ref):
    o_ref[...] = _dvec(deg_ref) * jnp.dot(
        x_ref[...], w_ref[...], preferred_element_type=jnp.float32)


def _tc_scale_matmul(deg, x, w):
    n, f = x.shape
    h = w.shape[1]
    grid = n // _BLK
    return pl.pallas_call(
        _scale_matmul_body,
        grid=(grid,),
        in_specs=[
            pl.BlockSpec((_BLK, 2), lambda i: (i, 0)),
            pl.BlockSpec((_BLK, f), lambda i: (i, 0)),
            pl.BlockSpec((f, h), lambda i: (0, 0)),
        ],
        out_specs=pl.BlockSpec((_BLK, h), lambda i: (i, 0)),
        out_shape=jax.ShapeDtypeStruct((n, h), jnp.float32),
    )(deg, x, w)


def _relu_matmul_body(deg_ref, a_ref, w_ref, o_ref):
    dv = _dvec(deg_ref)
    hidden = jnp.maximum(dv * (a_ref[0] + a_ref[1]), 0.0)
    o_ref[...] = dv * jnp.dot(
        hidden, w_ref[...], preferred_element_type=jnp.float32)


def _tc_relu_matmul(deg, agg, w, n):
    f = agg.shape[2]
    h = w.shape[1]
    grid = n // _BLK
    return pl.pallas_call(
        _relu_matmul_body,
        grid=(grid,),
        in_specs=[
            pl.BlockSpec((_BLK, 2), lambda i: (i, 0)),
            pl.BlockSpec((2, _BLK, f), lambda i: (0, i, 0)),
            pl.BlockSpec((f, h), lambda i: (0, 0)),
        ],
        out_specs=pl.BlockSpec((_BLK, h), lambda i: (i, 0)),
        out_shape=jax.ShapeDtypeStruct((n, h), jnp.float32),
    )(deg, agg, w)


def _encode_body(c, deg_ref, a_ref, o_ref):
    z = _dvec(deg_ref) * (a_ref[0] + a_ref[1])
    o_ref[...] = z[:, :c]


def _tc_encode(deg, agg, n, c):
    cp = agg.shape[2]
    grid = n // _BLK
    return pl.pallas_call(
        functools.partial(_encode_body, c),
        grid=(grid,),
        in_specs=[
            pl.BlockSpec((_BLK, 2), lambda i: (i, 0)),
            pl.BlockSpec((2, _BLK, cp), lambda i: (0, i, 0)),
        ],
        out_specs=pl.BlockSpec((_BLK, c), lambda i: (i, 0)),
        out_shape=jax.ShapeDtypeStruct((n, c), jnp.float32),
    )(deg, agg)


def _decode_body(zi_ref, zj_ref, o_ref):
    zz = lax.dot_general(
        zi_ref[...], zj_ref[...], (((1,), (1,)), ((), ())),
        preferred_element_type=jnp.float32)
    o_ref[...] = 1.0 / (1.0 + jnp.exp(-zz))


def _tc_decode(z):
    n, c = z.shape
    blk = 2048
    grid = pl.cdiv(n, blk)
    return pl.pallas_call(
        _decode_body,
        grid=(grid, grid),
        in_specs=[
            pl.BlockSpec((blk, c), lambda i, j: (i, 0)),
            pl.BlockSpec((blk, c), lambda i, j: (j, 0)),
        ],
        out_specs=pl.BlockSpec((blk, blk), lambda i, j: (i, j)),
        out_shape=jax.ShapeDtypeStruct((n, n), jnp.float32),
    )(z, z)


def kernel(x, edge_index, W1, W2):
    n, _ = x.shape
    e = edge_index.shape[1]
    row = edge_index[0]
    col = edge_index[1]
    code = W2.shape[1]
    if W2.shape[1] % 128:
        W2 = jnp.pad(W2, ((0, 0), (0, 128 - W2.shape[1] % 128)))
    info = plsc.get_sparse_core_info()
    nw = info.num_cores * info.num_subcores
    e_pad = pl.cdiv(e, nw * 16) * nw * 16
    if e_pad != e:
        spread = n + (jnp.arange(e_pad - e, dtype=jnp.int32) % 64)
        row = jnp.concatenate([row, spread])
        col = jnp.pad(col, (0, e_pad - e))
        e = e_pad

    deg = _make_degree(n, e)(row).T
    xf1 = _tc_scale_matmul(deg, x, W1)
    agg1 = _make_edge_aggregate(n, e, W1.shape[1])(xf1, col, row)
    xf2 = _tc_relu_matmul(deg, agg1, W2, n)
    agg2 = _make_edge_aggregate(n, e, W2.shape[1])(xf2, col, row)
    z = _tc_encode(deg, agg2, n, code)
    return _tc_decode(z)
